# baseline (speedup 1.0000x reference)
import functools

import jax
import jax.numpy as jnp
from jax import lax
from jax.experimental import pallas as pl
from jax.experimental.pallas import tpu as pltpu
from jax.experimental.pallas import tpu_sc as plsc

EMB = 128
N_X = 10000
N_EA = 320000
NC, NS = 2, 16
NW = NC * NS

X_PAD = 10240
N_ROWS = X_PAD + X_PAD + N_EA
PER_W = N_ROWS // NW
CH = 112
NCH = PER_W // CH


def _start_gather(table, idx_v, c, rows_v, sem):
    pltpu.async_copy(table.at[idx_v.at[pl.ds(c * CH, CH)]], rows_v, sem)


def _wait_gather(table, rows_v, sem):
    pltpu.make_async_copy(table.at[pl.ds(0, CH)], rows_v, sem).wait()


def _start_out(rows_v, out, wbase, c, sem):
    pltpu.async_copy(rows_v, out.at[pl.ds(wbase + c * CH, CH)], sem)


def _wait_out(rows_v, out, wbase, sem):
    pltpu.make_async_copy(rows_v, out.at[pl.ds(wbase, CH)], sem).wait()


def _body(idx_hbm, table_hbm, out_hbm, idx_v, rows0, rows1, g0, g1, o0, o1):
    wid = lax.axis_index("s") * NC + lax.axis_index("c")
    wbase = wid * PER_W
    pltpu.sync_copy(idx_hbm.at[pl.ds(wbase, PER_W)], idx_v)

    _start_gather(table_hbm, idx_v, 0, rows0, g0)
    _start_gather(table_hbm, idx_v, 1, rows1, g1)

    def step(k, carry):
        c = 2 * k
        _wait_gather(table_hbm, rows0, g0)
        _start_out(rows0, out_hbm, wbase, c, o0)
        _wait_gather(table_hbm, rows1, g1)
        _start_out(rows1, out_hbm, wbase, c + 1, o1)
        _wait_out(rows0, out_hbm, wbase, o0)
        _start_gather(table_hbm, idx_v, c + 2, rows0, g0)
        _wait_out(rows1, out_hbm, wbase, o1)
        _start_gather(table_hbm, idx_v, c + 3, rows1, g1)
        return carry

    lax.fori_loop(0, NCH // 2 - 1, step, 0)

    _wait_gather(table_hbm, rows0, g0)
    _start_out(rows0, out_hbm, wbase, NCH - 3, o0)
    _wait_gather(table_hbm, rows1, g1)
    _start_out(rows1, out_hbm, wbase, NCH - 2, o1)
    _wait_out(rows0, out_hbm, wbase, o0)
    _start_gather(table_hbm, idx_v, NCH - 1, rows0, g0)
    _wait_out(rows1, out_hbm, wbase, o1)
    _wait_gather(table_hbm, rows0, g0)
    pltpu.sync_copy(rows0, out_hbm.at[pl.ds(wbase + (NCH - 1) * CH, CH)])


_gather_all = functools.partial(
    pl.kernel,
    out_type=jax.ShapeDtypeStruct((N_ROWS, EMB), jnp.float32),
    scratch_types=[
        pltpu.VMEM((PER_W,), jnp.int32),
        pltpu.VMEM((CH, EMB), jnp.float32),
        pltpu.VMEM((CH, EMB), jnp.float32),
        pltpu.SemaphoreType.DMA,
        pltpu.SemaphoreType.DMA,
        pltpu.SemaphoreType.DMA,
        pltpu.SemaphoreType.DMA,
    ],
    mesh=plsc.VectorSubcoreMesh(core_axis_name="c", subcore_axis_name="s"),
)(_body)


def kernel(x, edge_attr, tuplefeat, x_table, ea_table, tf_table):
    pair_table = jnp.concatenate(
        [jnp.repeat(tf_table, 16, axis=0), jnp.tile(tf_table, (16, 1))], axis=1)
    table = jnp.concatenate([x_table, pair_table, ea_table], axis=0)

    tf = tuplefeat.astype(jnp.int32)
    tf_i = tf[:, 0] * 16 + tf[:, 1] + 32
    pad = jnp.zeros((X_PAD - N_X,), jnp.int32)
    idx = jnp.concatenate([
        x.reshape(-1).astype(jnp.int32), pad,
        tf_i, pad,
        edge_attr.astype(jnp.int32) + 32 + 256,
    ])

    out = _gather_all(idx, table)
    return (out[:N_X],
            out[2 * X_PAD:],
            out[X_PAD:X_PAD + N_X])
